# baseline (speedup 1.0000x reference)
import functools

import jax
import jax.numpy as jnp
from jax import lax
from jax.experimental import pallas as pl
from jax.experimental.pallas import tpu as pltpu
from jax.experimental.pallas import tpu_sc as plsc


def _pooled_lookup(B, T, D):
    info = plsc.get_sparse_core_info()
    NC, NS, L = info.num_cores, info.num_subcores, info.num_lanes
    NW = NC * NS
    assert B % NW == 0 and D % L == 0 and D // L == 4
    BPW = B // NW
    G = (T + L - 1) // L
    T_pad = G * L
    TAIL = T - (G - 1) * L
    C0 = 104
    C1 = T - C0
    mesh = plsc.VectorSubcoreMesh(core_axis_name="c", subcore_axis_name="s")

    @functools.partial(
        pl.kernel,
        mesh=mesh,
        compiler_params=pltpu.CompilerParams(use_tc_tiling_on_sc=False),
        out_type=jax.ShapeDtypeStruct((B * D,), jnp.float32),
        scratch_types=[
            pltpu.VMEM((T,), jnp.int32),
            pltpu.VMEM((T_pad,), jnp.float32),
            pltpu.VMEM((T_pad, D), jnp.float32),
            pltpu.VMEM((D,), jnp.float32),
            pltpu.SemaphoreType.DMA,
        ],
    )
    def k(tok_hbm, msk_hbm, table_hbm, out_hbm, idx_v, mv_v, rows_v, orow_v, sem):
        wid = lax.axis_index("s") * NC + lax.axis_index("c")
        z = jnp.zeros((L,), jnp.float32)
        for r in range(T, T_pad):
            for kk in range(4):
                rows_v[r, pl.ds(kk * L, L)] = z
        lane = lax.iota(jnp.int32, 16)

        def row_body(j, carry):
            b = wid * BPW + j
            pltpu.sync_copy(tok_hbm.at[pl.ds(b * T, T)], idx_v)
            pltpu.sync_copy(msk_hbm.at[pl.ds(b * T, T)], mv_v.at[pl.ds(0, T)])
            g1 = pltpu.async_copy(
                table_hbm.at[idx_v.at[pl.ds(0, C0)]], rows_v.at[pl.ds(0, C0)], sem
            )
            g2 = pltpu.async_copy(
                table_hbm.at[idx_v.at[pl.ds(C0, C1)]], rows_v.at[pl.ds(C0, C1)], sem
            )
            mtail = mv_v[pl.ds((G - 1) * L, L)]
            mv_v[pl.ds((G - 1) * L, L)] = jnp.where(lane < TAIL, mtail, 0.0)
            g1.wait()
            g2.wait()

            def red(g, acc):
                a0, a1, a2, a3, cntv = acc
                base = g * L
                mvec = mv_v[pl.ds(base, L)]
                cntv = cntv + mvec
                for i in range(L):
                    m = mvec[i]
                    t = base + i
                    a0 = a0 + rows_v[t, pl.ds(0, L)] * m
                    a1 = a1 + rows_v[t, pl.ds(L, L)] * m
                    a2 = a2 + rows_v[t, pl.ds(2 * L, L)] * m
                    a3 = a3 + rows_v[t, pl.ds(3 * L, L)] * m
                return (a0, a1, a2, a3, cntv)

            a0, a1, a2, a3, cntv = lax.fori_loop(0, G, red, (z, z, z, z, z))
            cnt = cntv[0]
            for i in range(1, L):
                cnt = cnt + cntv[i]
            denom = jnp.maximum(jnp.zeros((L,), jnp.float32) + cnt, 1.0)
            orow_v[pl.ds(0, L)] = a0 / denom
            orow_v[pl.ds(L, L)] = a1 / denom
            orow_v[pl.ds(2 * L, L)] = a2 / denom
            orow_v[pl.ds(3 * L, L)] = a3 / denom
            pltpu.sync_copy(orow_v, out_hbm.at[pl.ds(b * D, D)])
            return carry

        lax.fori_loop(0, BPW, row_body, 0)

    return k


def kernel(token_ids, mask, table):
    B, T = token_ids.shape
    V, D = table.shape
    tok_flat = token_ids.astype(jnp.int32).reshape(-1)
    mask_flat = mask.astype(jnp.float32).reshape(-1)
    out_flat = _pooled_lookup(B, T, D)(tok_flat, mask_flat, table)
    return out_flat.reshape(B, D)


# speedup vs baseline: 1.0935x; 1.0935x over previous
import functools

import jax
import jax.numpy as jnp
from jax import lax
from jax.experimental import pallas as pl
from jax.experimental.pallas import tpu as pltpu
from jax.experimental.pallas import tpu_sc as plsc


def _pooled_lookup(B, T, D):
    info = plsc.get_sparse_core_info()
    NC, NS, L = info.num_cores, info.num_subcores, info.num_lanes
    NW = NC * NS
    assert B % NW == 0 and D % L == 0 and D // L == 4
    BPW = B // NW
    assert BPW % 2 == 0
    G = (T + L - 1) // L
    TAIL = T - (G - 1) * L
    C0 = 104
    C1 = T - C0
    SLAB = BPW * T
    mesh = plsc.VectorSubcoreMesh(core_axis_name="c", subcore_axis_name="s")

    @functools.partial(
        pl.kernel,
        mesh=mesh,
        compiler_params=pltpu.CompilerParams(use_tc_tiling_on_sc=False),
        out_type=jax.ShapeDtypeStruct((B * D,), jnp.float32),
        scratch_types=[
            pltpu.VMEM((SLAB,), jnp.int32),
            pltpu.VMEM((SLAB + L,), jnp.float32),
            pltpu.VMEM((2, T, D), jnp.float32),
            pltpu.VMEM((BPW * D,), jnp.float32),
            pltpu.SemaphoreType.DMA,
            pltpu.SemaphoreType.DMA,
        ],
    )
    def k(tok_hbm, msk_hbm, table_hbm, out_hbm, tok_v, msk_v, rows_v, out_v, sem0, sem1):
        wid = lax.axis_index("s") * NC + lax.axis_index("c")
        slab_base = wid * SLAB
        pltpu.sync_copy(tok_hbm.at[pl.ds(slab_base, SLAB)], tok_v.at[pl.ds(0, SLAB)])
        pltpu.sync_copy(msk_hbm.at[pl.ds(slab_base, SLAB)], msk_v.at[pl.ds(0, SLAB)])
        lane = lax.iota(jnp.int32, L)
        z = jnp.zeros((L,), jnp.float32)
        sems = (sem0, sem1)

        def issue(j, buf_i, sem):
            base = j * T
            pltpu.async_copy(
                table_hbm.at[tok_v.at[pl.ds(base, C0)]],
                rows_v.at[buf_i].at[pl.ds(0, C0)],
                sem,
            )
            pltpu.async_copy(
                table_hbm.at[tok_v.at[pl.ds(base + C0, C1)]],
                rows_v.at[buf_i].at[pl.ds(C0, C1)],
                sem,
            )

        def drain(buf_i, sem):
            pltpu.make_async_copy(
                table_hbm.at[pl.ds(0, C0)], rows_v.at[buf_i].at[pl.ds(0, C0)], sem
            ).wait()
            pltpu.make_async_copy(
                table_hbm.at[pl.ds(0, C1)], rows_v.at[buf_i].at[pl.ds(C0, C1)], sem
            ).wait()

        def reduce_row(j, buf_i):
            buf = rows_v.at[buf_i]
            base = j * T
            a = [z, z, z, z]
            cntv = z
            for g in range(G):
                mvec = msk_v[pl.ds(base + g * L, L)]
                nv = L
                if g == G - 1:
                    mvec = jnp.where(lane < TAIL, mvec, 0.0)
                    nv = TAIL
                cntv = cntv + mvec
                for i in range(nv):
                    t = g * L + i
                    m = mvec[i]
                    for kk in range(4):
                        a[kk] = a[kk] + buf[t, pl.ds(kk * L, L)] * m
            cnt = cntv[0]
            for i in range(1, L):
                cnt = cnt + cntv[i]
            denom = jnp.maximum(z + cnt, 1.0)
            for kk in range(4):
                out_v[pl.ds(j * D + kk * L, L)] = a[kk] / denom

        issue(0, 0, sem0)
        issue(1, 1, sem1)

        def step(s, carry):
            for half in range(2):
                j = 2 * s + half
                drain(half, sems[half])
                reduce_row(j, half)

                @pl.when(s < BPW // 2 - 1)
                def _():
                    issue(j + 2, half, sems[half])

            return carry

        lax.fori_loop(0, BPW // 2, step, 0)
        pltpu.sync_copy(out_v, out_hbm.at[pl.ds(wid * BPW * D, BPW * D)])

    return k


def kernel(token_ids, mask, table):
    B, T = token_ids.shape
    V, D = table.shape
    tok_flat = token_ids.astype(jnp.int32).reshape(-1)
    mask_flat = mask.astype(jnp.float32).reshape(-1)
    out_flat = _pooled_lookup(B, T, D)(tok_flat, mask_flat, table)
    return out_flat.reshape(B, D)
